# Initial kernel scaffold; baseline (speedup 1.0000x reference)
import jax
import jax.numpy as jnp
from jax.experimental import pallas as pl


def kernel(x, layer1):
    raise NotImplementedError("write your pallas kernel here")


# speedup vs baseline: 42.8993x; 42.8993x over previous
import functools

import jax
import jax.numpy as jnp
from jax import lax
from jax.experimental import pallas as pl
from jax.experimental.pallas import tpu as pltpu
from jax.experimental.pallas import tpu_sc as plsc

_info = plsc.get_sparse_core_info()
_NC, _NS, _L = _info.num_cores, _info.num_subcores, _info.num_lanes
_NW = _NC * _NS

_TEX_W = 66
_TEX_SIZE = _TEX_W * _TEX_W

_UNROLL = 4


def _make_sc_kernel(n_points):
    assert n_points % (_NW * _L * _UNROLL) == 0
    chunk = n_points // _NW
    mesh = plsc.VectorSubcoreMesh(core_axis_name="c", subcore_axis_name="s")

    @functools.partial(
        pl.kernel,
        mesh=mesh,
        out_type=jax.ShapeDtypeStruct((n_points,), jnp.float32),
        scratch_types=[
            pltpu.VMEM((2 * chunk,), jnp.float32),
            pltpu.VMEM((_TEX_SIZE,), jnp.float32),
            pltpu.VMEM((chunk,), jnp.float32),
        ],
        compiler_params=pltpu.CompilerParams(needs_layout_passes=False),
    )
    def sc_kernel(x_hbm, tex_hbm, out_hbm, xv, texv, outv):
        wid = lax.axis_index("s") * _NC + lax.axis_index("c")
        base = wid * chunk
        pltpu.sync_copy(tex_hbm, texv)
        pltpu.sync_copy(x_hbm.at[pl.ds(2 * base, 2 * chunk)], xv)

        iota = lax.iota(jnp.int32, _L)
        iota2 = iota * 2

        def body(i, carry):
            off0 = i * (_L * _UNROLL)
            for u in range(_UNROLL):
                off = off0 + u * _L
                idxx = iota2 + 2 * off
                gx = plsc.load_gather(xv, [idxx])
                gy = plsc.load_gather(xv, [idxx + 1])
                ux = gx * 64.0 + 63.5
                uy = gy * 64.0 + 63.5
                jx = ux.astype(jnp.int32)
                jy = uy.astype(jnp.int32)
                fx = ux - jx.astype(jnp.float32)
                fy = uy - jy.astype(jnp.float32)
                cx0 = jx - 63
                row0 = (jy - 63) * _TEX_W
                f00 = row0 + cx0
                f10 = f00 + _TEX_W
                v00 = plsc.load_gather(texv, [f00])
                v01 = plsc.load_gather(texv, [f00 + 1])
                v10 = plsc.load_gather(texv, [f10])
                v11 = plsc.load_gather(texv, [f10 + 1])
                wx0 = 1.0 - fx
                t0 = v00 * wx0 + v01 * fx
                t1 = v10 * wx0 + v11 * fx
                outv[pl.ds(off, _L)] = t0 * (1.0 - fy) + t1 * fy
            return carry

        lax.fori_loop(0, chunk // (_L * _UNROLL), body, 0)
        pltpu.sync_copy(outv, out_hbm.at[pl.ds(base, chunk)])

    return sc_kernel


def kernel(x, layer1):
    batch, h, w, _ = x.shape
    n_points = batch * h * w
    xf = x.reshape(-1)
    texp = jnp.pad(layer1[0, 0], 1).reshape(-1)
    out = _make_sc_kernel(n_points)(xf, texp)
    return out.reshape(batch, 1, h, w)


# speedup vs baseline: 812.4813x; 18.9393x over previous
import functools

import jax
import jax.numpy as jnp
from jax import lax
from jax.experimental import pallas as pl
from jax.experimental.pallas import tpu as pltpu
from jax.experimental.pallas import tpu_sc as plsc

_info = plsc.get_sparse_core_info()
_NC, _NS, _L = _info.num_cores, _info.num_subcores, _info.num_lanes
_NW = _NC * _NS

_TEX_H = 72
_TEX_W = 128
_LANE = 128
_SUB = 8


def _make_sc_kernel(batch, h, w):
    n_points = batch * h * w
    slabs = _NW // batch
    rows = h // slabs
    cblk = w // _LANE
    grp = _LANE // _L
    chunk = rows * w
    mesh = plsc.VectorSubcoreMesh(core_axis_name="c", subcore_axis_name="s")

    @functools.partial(
        pl.kernel,
        mesh=mesh,
        out_type=jax.ShapeDtypeStruct((n_points,), jnp.float32),
        scratch_types=[
            pltpu.VMEM((2 * chunk,), jnp.float32),
            pltpu.VMEM((_TEX_H * _TEX_W,), jnp.float32),
            pltpu.VMEM((chunk,), jnp.float32),
        ],
        compiler_params=pltpu.CompilerParams(needs_layout_passes=False),
    )
    def sc_kernel(x_hbm, tex_hbm, out_hbm, xv, texv, outv):
        wid = lax.axis_index("s") * _NC + lax.axis_index("c")
        base = wid * chunk
        pltpu.sync_copy(tex_hbm, texv)
        pltpu.sync_copy(x_hbm.at[pl.ds(2 * base, 2 * chunk)], xv)

        def row_body(r, carry):
            xrow = r * (2 * w)
            orow = (r >> 3) * (cblk * _SUB * _LANE) + (r & 7) * _LANE

            def blk_body(cb, carry2):
                xb = xrow + cb * (2 * _LANE)
                ob = orow + cb * (_SUB * _LANE)
                for k in range(grp):
                    gx = xv[pl.ds(xb + k * _L, _L)]
                    gy = xv[pl.ds(xb + _LANE + k * _L, _L)]
                    ux = gx * 64.0 + 63.5
                    uy = gy * 64.0 + 63.5
                    jx = ux.astype(jnp.int32)
                    jy = uy.astype(jnp.int32)
                    fx = ux - jx.astype(jnp.float32)
                    fy = uy - jy.astype(jnp.float32)
                    f00 = ((jy - 63) << 7) + (jx - 63)
                    f10 = f00 + _TEX_W
                    v00 = plsc.load_gather(texv, [f00])
                    v01 = plsc.load_gather(texv, [f00 + 1])
                    v10 = plsc.load_gather(texv, [f10])
                    v11 = plsc.load_gather(texv, [f10 + 1])
                    wx0 = 1.0 - fx
                    t0 = v00 * wx0 + v01 * fx
                    t1 = v10 * wx0 + v11 * fx
                    outv[pl.ds(ob + k * _L, _L)] = t0 * (1.0 - fy) + t1 * fy
                return carry2

            lax.fori_loop(0, cblk, blk_body, 0)
            return carry

        lax.fori_loop(0, rows, row_body, 0)
        pltpu.sync_copy(outv, out_hbm.at[pl.ds(base, chunk)])

    return sc_kernel


def kernel(x, layer1):
    batch, h, w, _ = x.shape
    xflat = (
        x.reshape(batch, h, w // _LANE, _LANE, 2)
        .transpose(0, 1, 2, 4, 3)
        .reshape(-1)
    )
    texp = jnp.pad(layer1[0, 0], ((1, _TEX_H - 65), (1, _TEX_W - 65)))
    out = _make_sc_kernel(batch, h, w)(xflat, texp.reshape(-1))
    return (
        out.reshape(batch, h // _SUB, w // _LANE, _SUB, _LANE)
        .transpose(0, 1, 3, 2, 4)
        .reshape(batch, 1, h, w)
    )
